# baseline (speedup 1.0000x reference)
import functools
import math

import jax
import jax.numpy as jnp
from jax import lax
from jax.experimental import pallas as pl
from jax.experimental.pallas import tpu as pltpu
from jax.experimental.pallas import tpu_sc as plsc

EMBEDDING_DIM = 128
MAX_LEN = 8192
BATCH = 16384
HALF = EMBEDDING_DIM // 2

_info = plsc.get_sparse_core_info()
_NC, _NS = _info.num_cores, _info.num_subcores
_NW = _NC * _NS
_BPW = BATCH // _NW
_G = _BPW // 16


def _tables():
    div = jnp.exp(
        jnp.arange(0, EMBEDDING_DIM, 2, dtype=jnp.float32)
        * (-math.log(10000.0) / EMBEDDING_DIM)
    )
    coarse = (jnp.arange(128, dtype=jnp.float32) * 64.0)[:, None] * div
    fine = jnp.arange(64, dtype=jnp.float32)[:, None] * div
    return (
        jnp.sin(coarse).reshape(-1),
        jnp.cos(coarse).reshape(-1),
        jnp.sin(fine).reshape(-1),
        jnp.cos(fine).reshape(-1),
    )


@functools.partial(
    pl.kernel,
    mesh=plsc.VectorSubcoreMesh(core_axis_name="c", subcore_axis_name="s"),
    out_type=jax.ShapeDtypeStruct((BATCH, EMBEDDING_DIM), jnp.float32),
    compiler_params=pltpu.CompilerParams(needs_layout_passes=False),
    scratch_types=[
        pltpu.VMEM((_BPW,), jnp.int32),
        pltpu.VMEM((128 * HALF,), jnp.float32),
        pltpu.VMEM((128 * HALF,), jnp.float32),
        pltpu.VMEM((64 * HALF,), jnp.float32),
        pltpu.VMEM((64 * HALF,), jnp.float32),
        pltpu.VMEM((_BPW, EMBEDDING_DIM), jnp.float32),
    ],
)
def _pe_lookup(sa_hbm, ca_hbm, sb_hbm, cb_hbm, idx_hbm, out_hbm,
               idx_v, sa_v, ca_v, sb_v, cb_v, out_v):
    wid = lax.axis_index("s") * _NC + lax.axis_index("c")
    base = wid * _BPW
    pltpu.sync_copy(idx_hbm.at[pl.ds(base, _BPW)], idx_v)
    pltpu.sync_copy(sa_hbm, sa_v)
    pltpu.sync_copy(ca_hbm, ca_v)
    pltpu.sync_copy(sb_hbm, sb_v)
    pltpu.sync_copy(cb_hbm, cb_v)

    iota = lax.iota(jnp.int32, 16)

    def group(g, carry):
        t = idx_v[pl.ds(g * 16, 16)]
        aoff = (t >> 6) * HALF
        boff = (t & 63) * HALF
        rows = g * 16 + iota
        for k in range(HALF):
            ia = aoff + k
            ib = boff + k
            sa = plsc.load_gather(sa_v, [ia])
            ca = plsc.load_gather(ca_v, [ia])
            sb = plsc.load_gather(sb_v, [ib])
            cb = plsc.load_gather(cb_v, [ib])
            outs = sa * cb + ca * sb
            outc = ca * cb - sa * sb
            col = jnp.full((16,), 2 * k, jnp.int32)
            plsc.store_scatter(out_v, [rows, col], outs)
            plsc.store_scatter(out_v, [rows, col + 1], outc)
        return carry

    lax.fori_loop(0, _G, group, 0)
    pltpu.sync_copy(out_v, out_hbm.at[pl.ds(base, _BPW)])


def kernel(timesteps):
    sa, ca, sb, cb = _tables()
    return _pe_lookup(sa, ca, sb, cb, timesteps.astype(jnp.int32))


# speedup vs baseline: 3.1255x; 3.1255x over previous
import functools
import math

import jax
import jax.numpy as jnp
from jax import lax
from jax.experimental import pallas as pl
from jax.experimental.pallas import tpu as pltpu
from jax.experimental.pallas import tpu_sc as plsc

EMBEDDING_DIM = 128
MAX_LEN = 8192
BATCH = 16384

_info = plsc.get_sparse_core_info()
_NC, _NS = _info.num_cores, _info.num_subcores
_NW = _NC * _NS
_BPW = BATCH // _NW
_TROWS = MAX_LEN // _NS


def _pe_table() -> jnp.ndarray:
    position = jnp.arange(MAX_LEN, dtype=jnp.float32).reshape(-1, 1)
    div_term = jnp.exp(
        jnp.arange(0, EMBEDDING_DIM, 2, dtype=jnp.float32)
        * (-math.log(10000.0) / EMBEDDING_DIM)
    )
    ang = position * div_term
    return jnp.stack([jnp.sin(ang), jnp.cos(ang)], axis=-1).reshape(
        MAX_LEN, EMBEDDING_DIM
    )


@functools.partial(
    pl.kernel,
    mesh=plsc.VectorSubcoreMesh(core_axis_name="c", subcore_axis_name="s"),
    out_type=jax.ShapeDtypeStruct((BATCH, EMBEDDING_DIM), jnp.float32),
    scratch_types=[
        pltpu.VMEM((_BPW,), jnp.int32),
        pltpu.VMEM((_BPW, EMBEDDING_DIM), jnp.float32),
        pltpu.VMEM_SHARED((MAX_LEN // 2, EMBEDDING_DIM), jnp.float32),
        pltpu.SemaphoreType.DMA,
    ],
)
def _gather(table_hbm, idx_hbm, out_hbm, idx_v, rows_v, table_sp, sem):
    sid = lax.axis_index("s")
    wid = sid * _NC + lax.axis_index("c")
    base = wid * _BPW
    pltpu.sync_copy(
        table_hbm.at[pl.ds(sid * (_TROWS // 2), _TROWS // 2)],
        table_sp.at[pl.ds(sid * (_TROWS // 2), _TROWS // 2)],
    )
    pltpu.sync_copy(idx_hbm.at[pl.ds(base, _BPW)], idx_v)
    for i in range(_BPW // 16):
        idx_v[pl.ds(i * 16, 16)] = idx_v[pl.ds(i * 16, 16)] & 4095
    plsc.subcore_barrier()
    pltpu.async_copy(table_sp.at[idx_v], rows_v, sem).wait()
    pltpu.sync_copy(rows_v, out_hbm.at[pl.ds(base, _BPW)])


def kernel(timesteps):
    table = _pe_table()
    return _gather(table, timesteps.astype(jnp.int32))


# speedup vs baseline: 3.2767x; 1.0484x over previous
import functools
import math

import jax
import jax.numpy as jnp
from jax import lax
from jax.experimental import pallas as pl
from jax.experimental.pallas import tpu as pltpu
from jax.experimental.pallas import tpu_sc as plsc

EMBEDDING_DIM = 128
MAX_LEN = 8192
BATCH = 16384

_info = plsc.get_sparse_core_info()
_NC, _NS = _info.num_cores, _info.num_subcores
_NW = _NC * _NS
_BPW = BATCH // _NW


def _pe_table() -> jnp.ndarray:
    position = jnp.arange(MAX_LEN, dtype=jnp.float32).reshape(-1, 1)
    div_term = jnp.exp(
        jnp.arange(0, EMBEDDING_DIM, 2, dtype=jnp.float32)
        * (-math.log(10000.0) / EMBEDDING_DIM)
    )
    ang = position * div_term
    return jnp.stack([jnp.sin(ang), jnp.cos(ang)], axis=-1).reshape(
        MAX_LEN, EMBEDDING_DIM
    )


@functools.partial(
    pl.kernel,
    mesh=plsc.VectorSubcoreMesh(core_axis_name="c", subcore_axis_name="s"),
    out_type=jax.ShapeDtypeStruct((BATCH, EMBEDDING_DIM), jnp.float32),
    compiler_params=pltpu.CompilerParams(
        disable_bounds_checks=True,
        disable_semaphore_checks=True,
        skip_device_barrier=True,
    ),
    scratch_types=[
        pltpu.VMEM((_BPW,), jnp.int32),
        pltpu.VMEM((_BPW, EMBEDDING_DIM), jnp.float32),
        pltpu.SemaphoreType.DMA,
    ],
)
def _gather(table_hbm, idx_hbm, out_hbm, idx_v, rows_v, sem):
    wid = lax.axis_index("s") * _NC + lax.axis_index("c")
    base = wid * _BPW
    pltpu.sync_copy(idx_hbm.at[pl.ds(base, _BPW)], idx_v)
    pltpu.async_copy(table_hbm.at[idx_v], rows_v, sem).wait()
    pltpu.sync_copy(rows_v, out_hbm.at[pl.ds(base, _BPW)])


def kernel(timesteps):
    table = _pe_table()
    return _gather(table, timesteps.astype(jnp.int32))
